# baseline (device time: 113427 ns/iter reference)
import jax
import jax.numpy as jnp
from jax import lax
from jax.experimental import pallas as pl
from jax.experimental.pallas import tpu as pltpu

N_DEV = 4
B, SQ, SKV, D_MODEL = 2, 512, 512, 768
HQ, DH = 8, 64
DQ = HQ * DH
BLK = 64


def kernel(x, Wq, K_ext, V_ext, Wo):
    def body(x_ref, wq_ref, k_hbm, v_hbm, wo_ref, out_ref,
             k_vmem, v_vmem, ctx_vmem, comm_ref,
             copy_sems, send_sems, recv_sems):
        my = lax.axis_index("i")
        left = (my + N_DEV - 1) % N_DEV
        right = (my + 1) % N_DEV

        barrier = pltpu.get_barrier_semaphore()
        for nbr in (left, right):
            pl.semaphore_signal(barrier, inc=1, device_id=(nbr,),
                                device_id_type=pl.DeviceIdType.MESH)
        pl.semaphore_wait(barrier, 2)

        h0 = my * HQ
        kcp = pltpu.make_async_copy(
            k_hbm.at[:, :, pl.ds(h0, HQ), :], k_vmem, copy_sems.at[0])
        vcp = pltpu.make_async_copy(
            v_hbm.at[:, :, pl.ds(h0, HQ), :], v_vmem, copy_sems.at[1])
        kcp.start()
        vcp.start()

        qb = lax.broadcasted_iota(jnp.int32, (SQ, SKV), 0) // BLK
        kb = lax.broadcasted_iota(jnp.int32, (SQ, SKV), 1) // BLK
        mask = (qb == kb) | ((kb % 4) == (qb % 4))

        wq_bf = wq_ref[...].astype(jnp.bfloat16)
        wo_bf = wo_ref[...].astype(jnp.bfloat16)

        kcp.wait()
        vcp.wait()

        for b in range(B):
            xb = x_ref[b].astype(jnp.bfloat16)
            q_all = jnp.dot(xb, wq_bf,
                            preferred_element_type=jnp.float32)
            for h in range(HQ):
                qh = q_all[:, h * DH:(h + 1) * DH].astype(jnp.bfloat16)
                kh = k_vmem[b, :, h, :].astype(jnp.bfloat16)
                vh = v_vmem[b, :, h, :].astype(jnp.bfloat16)
                scores = lax.dot_general(
                    qh, kh, (((1,), (1,)), ((), ())),
                    preferred_element_type=jnp.float32) * 0.125
                scores = jnp.where(mask, scores, -1e9)
                m = jnp.max(scores, axis=1, keepdims=True)
                w = jnp.exp(scores - m)
                w = w / jnp.sum(w, axis=1, keepdims=True)
                ctxh = jnp.dot(w.astype(jnp.bfloat16), vh,
                               preferred_element_type=jnp.float32)
                ctx_vmem[:, h * DH:(h + 1) * DH] = ctxh.astype(jnp.bfloat16)
            partial = jnp.dot(ctx_vmem[...], wo_bf,
                              preferred_element_type=jnp.float32)
            out_ref[b] = partial
            comm_ref[0, b] = partial.astype(jnp.bfloat16)

        for hop in range(N_DEV - 1):
            rdma = pltpu.make_async_remote_copy(
                src_ref=comm_ref.at[hop],
                dst_ref=comm_ref.at[hop + 1],
                send_sem=send_sems.at[hop],
                recv_sem=recv_sems.at[hop],
                device_id=(right,),
                device_id_type=pl.DeviceIdType.MESH,
            )
            rdma.start()
            rdma.wait()
            out_ref[...] = out_ref[...] + comm_ref[hop + 1].astype(jnp.float32)

    return pl.pallas_call(
        body,
        out_shape=jax.ShapeDtypeStruct((B, SQ, D_MODEL), jnp.float32),
        in_specs=[
            pl.BlockSpec(memory_space=pltpu.MemorySpace.VMEM),
            pl.BlockSpec(memory_space=pltpu.MemorySpace.VMEM),
            pl.BlockSpec(memory_space=pl.ANY),
            pl.BlockSpec(memory_space=pl.ANY),
            pl.BlockSpec(memory_space=pltpu.MemorySpace.VMEM),
        ],
        out_specs=pl.BlockSpec(memory_space=pltpu.MemorySpace.VMEM),
        scratch_shapes=[
            pltpu.VMEM((B, SQ, HQ, DH), jnp.float32),
            pltpu.VMEM((B, SQ, HQ, DH), jnp.float32),
            pltpu.VMEM((SQ, DQ), jnp.bfloat16),
            pltpu.VMEM((N_DEV, B, SQ, D_MODEL), jnp.bfloat16),
            pltpu.SemaphoreType.DMA((2,)),
            pltpu.SemaphoreType.DMA((N_DEV - 1,)),
            pltpu.SemaphoreType.DMA((N_DEV - 1,)),
        ],
        compiler_params=pltpu.CompilerParams(collective_id=0),
    )(x, Wq, K_ext, V_ext, Wo)


# device time: 90053 ns/iter; 1.2596x vs baseline; 1.2596x over previous
import jax
import jax.numpy as jnp
from jax import lax
from jax.experimental import pallas as pl
from jax.experimental.pallas import tpu as pltpu

N_DEV = 4
B, SQ, SKV, D_MODEL = 2, 512, 512, 768
HQ, DH = 8, 64
DQ = HQ * DH
BLK = 64
NGRP = 4
GRP = 2 * BLK

PERM = [0, 4, 1, 5, 2, 6, 3, 7]
INV_PERM = [0, 2, 4, 6, 1, 3, 5, 7]


def _permute_rows(a, order):
    return jnp.concatenate([a[b * BLK:(b + 1) * BLK] for b in order], axis=0)


def kernel(x, Wq, K_ext, V_ext, Wo):
    def body(x_ref, wq_ref, k_hbm, v_hbm, wo_ref, out_ref,
             k_vmem, v_vmem, ctx_vmem, comm_ref,
             copy_sems, send_sems, recv_sems):
        my = lax.axis_index("i")
        left = (my + N_DEV - 1) % N_DEV
        right = (my + 1) % N_DEV

        barrier = pltpu.get_barrier_semaphore()
        for nbr in (left, right):
            pl.semaphore_signal(barrier, inc=1, device_id=(nbr,),
                                device_id_type=pl.DeviceIdType.MESH)
        pl.semaphore_wait(barrier, 2)

        h0 = my * HQ
        cps = []
        for j, pb in enumerate(PERM):
            for t, (hbm, vm) in enumerate(((k_hbm, k_vmem), (v_hbm, v_vmem))):
                cp = pltpu.make_async_copy(
                    hbm.at[:, pl.ds(pb * BLK, BLK), pl.ds(h0, HQ), :],
                    vm.at[:, pl.ds(j * BLK, BLK)],
                    copy_sems.at[2 * j + t])
                cp.start()
                cps.append(cp)

        wq_bf = wq_ref[...].astype(jnp.bfloat16)
        wo_bf = wo_ref[...].astype(jnp.bfloat16)

        for cp in cps:
            cp.wait()

        for b in range(B):
            xb = _permute_rows(x_ref[b], PERM).astype(jnp.bfloat16)
            q_all = jnp.dot(xb, wq_bf,
                            preferred_element_type=jnp.float32)
            for h in range(HQ):
                qh = q_all[:, h * DH:(h + 1) * DH].astype(jnp.bfloat16)
                kh = k_vmem[b, :, h, :].astype(jnp.bfloat16)
                vh = v_vmem[b, :, h, :].astype(jnp.bfloat16)
                s = jnp.concatenate([
                    lax.dot_general(
                        qh[m * GRP:(m + 1) * GRP],
                        kh[m * GRP:(m + 1) * GRP],
                        (((1,), (1,)), ((), ())),
                        preferred_element_type=jnp.float32)
                    for m in range(NGRP)], axis=0) * 0.125
                mx = jnp.max(s, axis=1, keepdims=True)
                w = jnp.exp(s - mx)
                w = (w / jnp.sum(w, axis=1, keepdims=True)).astype(jnp.bfloat16)
                for m in range(NGRP):
                    ctxm = jnp.dot(w[m * GRP:(m + 1) * GRP],
                                   vh[m * GRP:(m + 1) * GRP],
                                   preferred_element_type=jnp.float32)
                    ctx_vmem[m * GRP:(m + 1) * GRP,
                             h * DH:(h + 1) * DH] = ctxm.astype(jnp.bfloat16)
            ctx_unperm = _permute_rows(ctx_vmem[...], INV_PERM)
            partial = jnp.dot(ctx_unperm, wo_bf,
                              preferred_element_type=jnp.float32)
            out_ref[b] = partial
            comm_ref[0, b] = partial.astype(jnp.bfloat16)

        p1 = my ^ 1
        p2 = 3 - my
        for s, partner in enumerate((p1, p2)):
            rdma = pltpu.make_async_remote_copy(
                src_ref=comm_ref.at[2 * s],
                dst_ref=comm_ref.at[2 * s + 1],
                send_sem=send_sems.at[s],
                recv_sem=recv_sems.at[s],
                device_id=(partner,),
                device_id_type=pl.DeviceIdType.MESH,
            )
            rdma.start()
            rdma.wait()
            acc = out_ref[...] + comm_ref[2 * s + 1].astype(jnp.float32)
            out_ref[...] = acc
            if s == 0:
                comm_ref[2] = acc.astype(jnp.bfloat16)

    return pl.pallas_call(
        body,
        out_shape=jax.ShapeDtypeStruct((B, SQ, D_MODEL), jnp.float32),
        in_specs=[
            pl.BlockSpec(memory_space=pltpu.MemorySpace.VMEM),
            pl.BlockSpec(memory_space=pltpu.MemorySpace.VMEM),
            pl.BlockSpec(memory_space=pl.ANY),
            pl.BlockSpec(memory_space=pl.ANY),
            pl.BlockSpec(memory_space=pltpu.MemorySpace.VMEM),
        ],
        out_specs=pl.BlockSpec(memory_space=pltpu.MemorySpace.VMEM),
        scratch_shapes=[
            pltpu.VMEM((B, SQ, HQ, DH), jnp.float32),
            pltpu.VMEM((B, SQ, HQ, DH), jnp.float32),
            pltpu.VMEM((SQ, DQ), jnp.bfloat16),
            pltpu.VMEM((4, B, SQ, D_MODEL), jnp.bfloat16),
            pltpu.SemaphoreType.DMA((16,)),
            pltpu.SemaphoreType.DMA((2,)),
            pltpu.SemaphoreType.DMA((2,)),
        ],
        compiler_params=pltpu.CompilerParams(collective_id=0),
    )(x, Wq, K_ext, V_ext, Wo)


# device time: 50412 ns/iter; 2.2500x vs baseline; 1.7863x over previous
import jax
import jax.numpy as jnp
from jax import lax
from jax.experimental import pallas as pl
from jax.experimental.pallas import tpu as pltpu

N_DEV = 4
B, SQ, SKV, D_MODEL = 2, 512, 512, 768
HQ, DH = 8, 64
DQ = HQ * DH
H_TOT = 32
BLK = 64
NGRP = 4
GRP = 2 * BLK
CH = SQ // 2
NC = 2 * B

PERM = [0, 4, 1, 5, 2, 6, 3, 7]
INV_PERM = [0, 2, 4, 6, 1, 3, 5, 7]


def _permute_rows(a, order):
    return jnp.concatenate([a[b * BLK:(b + 1) * BLK] for b in order], axis=0)


def kernel(x, Wq, K_ext, V_ext, Wo):
    def body(x_ref, wq_ref, k_hbm, v_hbm, wo_ref, out_ref,
             k_vmem, v_vmem, ctx_vmem, s1s, s1r, s2s, s2r,
             copy_sems, send_sems, recv_sems):
        my = lax.axis_index("i")
        p1 = my ^ 1
        p2 = 3 - my

        cps = []
        for j, pb in enumerate(PERM):
            for t, (hbm, vm) in enumerate(((k_hbm, k_vmem), (v_hbm, v_vmem))):
                cp = pltpu.make_async_copy(
                    hbm.at[:, pl.ds(pb * BLK, BLK), pl.ds(my * DQ, DQ)],
                    vm.at[:, pl.ds(j * BLK, BLK)],
                    copy_sems.at[2 * j + t])
                cp.start()
                cps.append(cp)

        barrier = pltpu.get_barrier_semaphore()
        for nbr in (p1, p2):
            pl.semaphore_signal(barrier, inc=1, device_id=(nbr,),
                                device_id_type=pl.DeviceIdType.MESH)
        pl.semaphore_wait(barrier, 2)

        wq_bf = wq_ref[...].astype(jnp.bfloat16)
        wo_bf = wo_ref[...].astype(jnp.bfloat16)

        for cp in cps:
            cp.wait()

        def exchange(idx, partner, sbuf, rbuf, c):
            rdma = pltpu.make_async_remote_copy(
                src_ref=sbuf.at[c], dst_ref=rbuf.at[c],
                send_sem=send_sems.at[idx], recv_sem=recv_sems.at[idx],
                device_id=(partner,), device_id_type=pl.DeviceIdType.MESH)
            rdma.start()
            return rdma

        x1 = [None] * NC
        x2 = [None] * NC
        for b in range(B):
            xb = _permute_rows(x_ref[b], PERM).astype(jnp.bfloat16)
            q_all = jnp.dot(xb, wq_bf,
                            preferred_element_type=jnp.float32)
            for h in range(HQ):
                qh = q_all[:, h * DH:(h + 1) * DH].astype(jnp.bfloat16)
                kh = k_vmem[b, :, h * DH:(h + 1) * DH].astype(jnp.bfloat16)
                vh = v_vmem[b, :, h * DH:(h + 1) * DH].astype(jnp.bfloat16)
                s = jnp.concatenate([
                    lax.dot_general(
                        qh[m * GRP:(m + 1) * GRP],
                        kh[m * GRP:(m + 1) * GRP],
                        (((1,), (1,)), ((), ())),
                        preferred_element_type=jnp.float32)
                    for m in range(NGRP)], axis=0) * 0.125
                mx = jnp.max(s, axis=1, keepdims=True)
                w = jnp.exp(s - mx)
                w = (w / jnp.sum(w, axis=1, keepdims=True)).astype(jnp.bfloat16)
                for m in range(NGRP):
                    ctxm = jnp.dot(w[m * GRP:(m + 1) * GRP],
                                   vh[m * GRP:(m + 1) * GRP],
                                   preferred_element_type=jnp.float32)
                    ctx_vmem[m * GRP:(m + 1) * GRP,
                             h * DH:(h + 1) * DH] = ctxm.astype(jnp.bfloat16)
            ctx_unperm = _permute_rows(ctx_vmem[...], INV_PERM)
            partial = jnp.dot(ctx_unperm, wo_bf,
                              preferred_element_type=jnp.float32)
            for half in range(2):
                c = 2 * b + half
                s1s[c] = partial[half * CH:(half + 1) * CH].astype(jnp.bfloat16)
                x1[c] = exchange(c, p1, s1s, s1r, c)

        for c in range(NC):
            x1[c].wait()
            s2s[c] = s1s[c] + s1r[c]
            x2[c] = exchange(NC + c, p2, s2s, s2r, c)
        for c in range(NC):
            x2[c].wait()
            b, half = divmod(c, 2)
            out_ref[b, half * CH:(half + 1) * CH] = (
                s2s[c].astype(jnp.float32) + s2r[c].astype(jnp.float32))

    return pl.pallas_call(
        body,
        out_shape=jax.ShapeDtypeStruct((B, SQ, D_MODEL), jnp.float32),
        in_specs=[
            pl.BlockSpec(memory_space=pltpu.MemorySpace.VMEM),
            pl.BlockSpec(memory_space=pltpu.MemorySpace.VMEM),
            pl.BlockSpec(memory_space=pl.ANY),
            pl.BlockSpec(memory_space=pl.ANY),
            pl.BlockSpec(memory_space=pltpu.MemorySpace.VMEM),
        ],
        out_specs=pl.BlockSpec(memory_space=pltpu.MemorySpace.VMEM),
        scratch_shapes=[
            pltpu.VMEM((B, SQ, DQ), jnp.float32),
            pltpu.VMEM((B, SQ, DQ), jnp.float32),
            pltpu.VMEM((SQ, DQ), jnp.bfloat16),
            pltpu.VMEM((NC, CH, D_MODEL), jnp.bfloat16),
            pltpu.VMEM((NC, CH, D_MODEL), jnp.bfloat16),
            pltpu.VMEM((NC, CH, D_MODEL), jnp.bfloat16),
            pltpu.VMEM((NC, CH, D_MODEL), jnp.bfloat16),
            pltpu.SemaphoreType.DMA((16,)),
            pltpu.SemaphoreType.DMA((2 * NC,)),
            pltpu.SemaphoreType.DMA((2 * NC,)),
        ],
        compiler_params=pltpu.CompilerParams(collective_id=0),
    )(x, Wq,
      K_ext.reshape(B, SKV, H_TOT * DH),
      V_ext.reshape(B, SKV, H_TOT * DH),
      Wo)


# device time: 44189 ns/iter; 2.5669x vs baseline; 1.1408x over previous
import jax
import jax.numpy as jnp
from jax import lax
from jax.experimental import pallas as pl
from jax.experimental.pallas import tpu as pltpu

N_DEV = 4
B, SQ, SKV, D_MODEL = 2, 512, 512, 768
HQ, DH = 8, 64
DQ = HQ * DH
BLK = 64
NGRP = 4
GRP = 2 * BLK
CH = SQ // 2
NC = 2 * B

PERM = [0, 4, 1, 5, 2, 6, 3, 7]
INV_PERM = [0, 2, 4, 6, 1, 3, 5, 7]


def kernel(x, Wq, K_ext, V_ext, Wo):
    i = lax.axis_index("i")
    xp = jnp.concatenate(
        [x[:, pb * BLK:(pb + 1) * BLK] for pb in PERM], axis=1
    ).astype(jnp.bfloat16)
    kv = []
    for t in (K_ext, V_ext):
        ts = lax.dynamic_slice_in_dim(t, i * HQ, HQ, axis=2)
        ts = ts.reshape(B, SKV, DQ)
        kv.append(jnp.concatenate(
            [ts[:, pb * BLK:(pb + 1) * BLK] for pb in PERM], axis=1
        ).astype(jnp.bfloat16))
    Kp, Vp = kv

    def body(x_ref, wq_ref, k_ref, v_ref, wo_ref, out_ref,
             ctx_vmem, s1s, s1r, s2s, s2r, send_sems, recv_sems):
        my = lax.axis_index("i")
        p1 = my ^ 1
        p2 = 3 - my

        barrier = pltpu.get_barrier_semaphore()
        for nbr in (p1, p2):
            pl.semaphore_signal(barrier, inc=1, device_id=(nbr,),
                                device_id_type=pl.DeviceIdType.MESH)
        pl.semaphore_wait(barrier, 2)

        def exchange(idx, partner, sbuf, rbuf, c):
            rdma = pltpu.make_async_remote_copy(
                src_ref=sbuf.at[c], dst_ref=rbuf.at[c],
                send_sem=send_sems.at[idx], recv_sem=recv_sems.at[idx],
                device_id=(partner,), device_id_type=pl.DeviceIdType.MESH)
            rdma.start()
            return rdma

        x1 = [None] * NC
        x2 = [None] * NC
        for b in range(B):
            q_all = (jnp.dot(x_ref[b], wq_ref[...],
                             preferred_element_type=jnp.float32)
                     * 0.125).astype(jnp.bfloat16)
            for h in range(HQ):
                qh = q_all[:, h * DH:(h + 1) * DH]
                kh = k_ref[b, :, h * DH:(h + 1) * DH]
                vh = v_ref[b, :, h * DH:(h + 1) * DH]
                s = jnp.concatenate([
                    lax.dot_general(
                        qh[m * GRP:(m + 1) * GRP],
                        kh[m * GRP:(m + 1) * GRP],
                        (((1,), (1,)), ((), ())),
                        preferred_element_type=jnp.float32)
                    for m in range(NGRP)], axis=0)
                mx = jnp.max(s, axis=1, keepdims=True)
                w = jnp.exp(s - mx)
                w = (w / jnp.sum(w, axis=1, keepdims=True)).astype(jnp.bfloat16)
                for m in range(NGRP):
                    ctxm = jnp.dot(w[m * GRP:(m + 1) * GRP],
                                   vh[m * GRP:(m + 1) * GRP],
                                   preferred_element_type=jnp.float32)
                    ctx_vmem[m * GRP:(m + 1) * GRP,
                             h * DH:(h + 1) * DH] = ctxm.astype(jnp.bfloat16)
            ctx_unperm = jnp.concatenate(
                [ctx_vmem[pj * BLK:(pj + 1) * BLK] for pj in INV_PERM], axis=0)
            partial = jnp.dot(ctx_unperm, wo_ref[...],
                              preferred_element_type=jnp.float32)
            for half in range(2):
                c = 2 * b + half
                s1s[c] = partial[half * CH:(half + 1) * CH].astype(jnp.bfloat16)
                x1[c] = exchange(c, p1, s1s, s1r, c)

        for c in range(NC):
            x1[c].wait()
            s2s[c] = s1s[c] + s1r[c]
            x2[c] = exchange(NC + c, p2, s2s, s2r, c)
        for c in range(NC):
            x2[c].wait()
            b, half = divmod(c, 2)
            out_ref[b, half * CH:(half + 1) * CH] = (
                s2s[c].astype(jnp.float32) + s2r[c].astype(jnp.float32))

    return pl.pallas_call(
        body,
        out_shape=jax.ShapeDtypeStruct((B, SQ, D_MODEL), jnp.float32),
        in_specs=[pl.BlockSpec(memory_space=pltpu.MemorySpace.VMEM)] * 5,
        out_specs=pl.BlockSpec(memory_space=pltpu.MemorySpace.VMEM),
        scratch_shapes=[
            pltpu.VMEM((SQ, DQ), jnp.bfloat16),
            pltpu.VMEM((NC, CH, D_MODEL), jnp.bfloat16),
            pltpu.VMEM((NC, CH, D_MODEL), jnp.bfloat16),
            pltpu.VMEM((NC, CH, D_MODEL), jnp.bfloat16),
            pltpu.VMEM((NC, CH, D_MODEL), jnp.bfloat16),
            pltpu.SemaphoreType.DMA((2 * NC,)),
            pltpu.SemaphoreType.DMA((2 * NC,)),
        ],
        compiler_params=pltpu.CompilerParams(collective_id=0),
    )(xp, Wq.astype(jnp.bfloat16), Kp, Vp, Wo.astype(jnp.bfloat16))
